# baseline (speedup 1.0000x reference)
import jax
import jax.numpy as jnp
from jax import lax
from jax.experimental import pallas as pl
from jax.experimental.pallas import tpu as pltpu
from jax.experimental.pallas import tpu_sc as plsc

HIDDEN = 1024
FF = 2048
E = 8
TOK = 2048
FFB = 512
NFF = FF // FFB

BM = 128
BM_SHIFT = 7
NBLK = (2 * TOK) // BM + E - 1
NBLKP = 48
NPAD = NBLK * BM

NC, NS = 2, 16
NW = NC * NS
TPW = TOK // NW
CH = 16


def _nt(a, b):
    return lax.dot_general(a, b, (((1,), (1,)), ((), ())),
                           preferred_element_type=jnp.float32)


def _top2(logits):
    ii = lax.broadcasted_iota(jnp.int32, logits.shape, 1)
    m1 = jnp.max(logits, axis=1, keepdims=True)
    idx1 = jnp.min(jnp.where(logits == m1, ii, E), axis=1, keepdims=True)
    mask1 = ii == idx1
    rest = jnp.where(mask1, -jnp.inf, logits)
    m2 = jnp.max(rest, axis=1, keepdims=True)
    idx2 = jnp.min(jnp.where(rest == m2, ii, E), axis=1, keepdims=True)
    mask2 = ii == idx2
    return mask1, mask2


def _meta_body(hs_ref, rw_ref, scores_ref, s1b_ref, s2b_ref,
               pos1_ref, pos2_ref, be_ref):
    x = hs_ref[...]
    logits = _nt(x, rw_ref[...])
    mask1, mask2 = _top2(logits)
    topmask = mask1 | mask2
    scores = jnp.where(topmask, jax.nn.sigmoid(logits), 0.0)
    scores_ref[...] = scores

    s1 = jnp.sum(jnp.where(mask1, scores, 0.0), axis=1, keepdims=True)
    s2 = jnp.sum(jnp.where(mask2, scores, 0.0), axis=1, keepdims=True)
    s1b_ref[...] = jnp.broadcast_to(s1, (TOK, 16))
    s2b_ref[...] = jnp.broadcast_to(s2, (TOK, 16))

    tm = topmask.astype(jnp.float32)
    ri = lax.broadcasted_iota(jnp.int32, (TOK, TOK), 0)
    ci = lax.broadcasted_iota(jnp.int32, (TOK, TOK), 1)
    tri = (ci < ri).astype(jnp.float32)
    ranks = lax.dot_general(tri, tm, (((1,), (0,)), ((), ())),
                            preferred_element_type=jnp.float32)

    counts = jnp.sum(tm, axis=0, keepdims=True).astype(jnp.int32)
    nblk_e = (counts + (BM - 1)) >> BM_SHIFT
    ri8 = lax.broadcasted_iota(jnp.int32, (E, E), 0)
    ci8 = lax.broadcasted_iota(jnp.int32, (E, E), 1)
    tri8 = (ri8 < ci8).astype(jnp.float32)
    po_blk = lax.dot_general(nblk_e.astype(jnp.float32), tri8,
                             (((1,), (0,)), ((), ())),
                             preferred_element_type=jnp.float32)
    po_rows = po_blk * float(BM)

    val = po_rows + ranks
    pos1 = jnp.sum(jnp.where(mask1, val, 0.0), axis=1, keepdims=True)
    pos2 = jnp.sum(jnp.where(mask2, val, 0.0), axis=1, keepdims=True)
    pos1_ref[...] = pos1.astype(jnp.int32)
    pos2_ref[...] = pos2.astype(jnp.int32)

    nb = jnp.sum(nblk_e, axis=1, keepdims=True)
    po_i = po_blk.astype(jnp.int32)
    bb = lax.broadcasted_iota(jnp.int32, (NBLKP, E), 0)
    be_raw = jnp.sum((po_i <= bb).astype(jnp.int32), axis=1,
                     keepdims=True) - 1
    be_raw = jnp.maximum(be_raw, 0)
    ie = lax.broadcasted_iota(jnp.int32, (1, E), 1)
    laste = jnp.max(jnp.where(nblk_e > 0, ie, 0), axis=1, keepdims=True)
    rowi = lax.broadcasted_iota(jnp.int32, (NBLKP, 1), 0)
    be = jnp.where(rowi < nb, be_raw, laste)
    be = jnp.where(rowi == NBLKP - 1, nb, be)
    be_ref[...] = be


def _shared_body(hs_ref, sg_ref, su_ref, sd_ref, base_ref, acc_ref):
    f = pl.program_id(0)
    x = hs_ref[...]
    h = jax.nn.silu(_nt(x, sg_ref[...])) * _nt(x, su_ref[...])
    part = _nt(h, sd_ref[...])

    @pl.when(f == 0)
    def _():
        acc_ref[...] = part

    @pl.when(f > 0)
    def _():
        acc_ref[...] += part

    @pl.when(f == NFF - 1)
    def _():
        base_ref[...] = acc_ref[...]


def _gmm_body(be_ref, xg_ref, wg_ref, wu_ref, wd_ref, y_ref):
    b = pl.program_id(0)

    @pl.when(b < be_ref[NBLKP - 1])
    def _():
        x = xg_ref[...]
        h = jax.nn.silu(_nt(x, wg_ref[0])) * _nt(x, wu_ref[0])
        y_ref[...] = _nt(h, wd_ref[0])


def _dispatch_body(hs_hbm, pos1_hbm, pos2_hbm, xg_hbm,
                   p1v, p2v, xrows, sem):
    wid = lax.axis_index("s") * NC + lax.axis_index("c")
    base = wid * TPW
    pltpu.sync_copy(pos1_hbm.at[pl.ds(base, TPW)], p1v)
    pltpu.sync_copy(pos2_hbm.at[pl.ds(base, TPW)], p2v)
    pltpu.sync_copy(hs_hbm.at[pl.ds(base, TPW)], xrows)
    c1 = pltpu.async_copy(xrows, xg_hbm.at[p1v], sem)
    c2 = pltpu.async_copy(xrows, xg_hbm.at[p2v], sem)
    c1.wait()
    c2.wait()


def _combine_body(base_hbm, y_hbm, pos1_hbm, pos2_hbm, s1b_hbm, s2b_hbm,
                  out_hbm, p1v, p2v, y1, y2, bb, s1v, s2v, sem, osem):
    wid = lax.axis_index("s") * NC + lax.axis_index("c")
    nch = TPW // CH

    def fetch(c, slot):
        tok0 = wid * TPW + c * CH
        pltpu.sync_copy(pos1_hbm.at[pl.ds(tok0, CH)], p1v.at[slot])
        pltpu.sync_copy(pos2_hbm.at[pl.ds(tok0, CH)], p2v.at[slot])
        g1 = pltpu.async_copy(y_hbm.at[p1v.at[slot]], y1.at[slot], sem)
        g2 = pltpu.async_copy(y_hbm.at[p2v.at[slot]], y2.at[slot], sem)
        gb = pltpu.async_copy(base_hbm.at[pl.ds(tok0, CH)], bb.at[slot], sem)
        pltpu.sync_copy(s1b_hbm.at[pl.ds(tok0, CH)], s1v.at[slot])
        pltpu.sync_copy(s2b_hbm.at[pl.ds(tok0, CH)], s2v.at[slot])
        return g1, g2, gb

    pend = fetch(0, 0)
    owaits = []
    for c in range(nch):
        slot = c % 2
        for d in pend:
            d.wait()
        if c + 1 < nch:
            nxt = fetch(c + 1, (c + 1) % 2)
        y1s, y2s, bbs = y1.at[slot], y2.at[slot], bb.at[slot]
        s1s, s2s = s1v.at[slot], s2v.at[slot]

        def tbody(t, carry):
            s1 = s1s[t, :]
            s2 = s2s[t, :]
            for j in range(HIDDEN // 16):
                sl = pl.ds(j * 16, 16)
                bbs[t, sl] = bbs[t, sl] + s1 * y1s[t, sl] + s2 * y2s[t, sl]
            return carry

        lax.fori_loop(0, CH, tbody, 0)
        tok0 = wid * TPW + c * CH
        owaits.append(pltpu.async_copy(bb.at[slot], out_hbm.at[pl.ds(tok0, CH)], osem))
        if len(owaits) >= 2:
            owaits.pop(0).wait()
        if c + 1 < nch:
            pend = nxt
    for d in owaits:
        d.wait()


def _sc_mesh():
    return plsc.VectorSubcoreMesh(core_axis_name="c", subcore_axis_name="s",
                                  num_cores=NC, num_subcores=NS)


def _make_dispatch():
    return pl.kernel(
        _dispatch_body,
        out_type=jax.ShapeDtypeStruct((NPAD, HIDDEN), jnp.float32),
        mesh=_sc_mesh(),
        scratch_types=[
            pltpu.VMEM((TPW,), jnp.int32),
            pltpu.VMEM((TPW,), jnp.int32),
            pltpu.VMEM((TPW, HIDDEN), jnp.float32),
            pltpu.SemaphoreType.DMA,
        ],
    )


def _make_combine():
    return pl.kernel(
        _combine_body,
        out_type=jax.ShapeDtypeStruct((TOK, HIDDEN), jnp.float32),
        mesh=_sc_mesh(),
        scratch_types=[
            pltpu.VMEM((2, CH), jnp.int32),
            pltpu.VMEM((2, CH), jnp.int32),
            pltpu.VMEM((2, CH, HIDDEN), jnp.float32),
            pltpu.VMEM((2, CH, HIDDEN), jnp.float32),
            pltpu.VMEM((2, CH, HIDDEN), jnp.float32),
            pltpu.VMEM((2, CH, 16), jnp.float32),
            pltpu.VMEM((2, CH, 16), jnp.float32),
            pltpu.SemaphoreType.DMA,
            pltpu.SemaphoreType.DMA,
        ],
    )


def kernel(hidden_states, router_w, gate_w, up_w, down_w,
           shared_gate_w, shared_up_w, shared_down_w):
    hs = hidden_states.reshape(-1, HIDDEN)

    scores, s1b, s2b, pos1, pos2, be = pl.pallas_call(
        _meta_body,
        out_shape=[
            jax.ShapeDtypeStruct((TOK, E), jnp.float32),
            jax.ShapeDtypeStruct((TOK, 16), jnp.float32),
            jax.ShapeDtypeStruct((TOK, 16), jnp.float32),
            jax.ShapeDtypeStruct((TOK, 1), jnp.int32),
            jax.ShapeDtypeStruct((TOK, 1), jnp.int32),
            jax.ShapeDtypeStruct((NBLKP, 1), jnp.int32),
        ],
        compiler_params=pltpu.CompilerParams(
            vmem_limit_bytes=128 * 1024 * 1024),
    )(hs, router_w)

    base = pl.pallas_call(
        _shared_body,
        grid=(NFF,),
        in_specs=[
            pl.BlockSpec((TOK, HIDDEN), lambda f: (0, 0)),
            pl.BlockSpec((FFB, HIDDEN), lambda f: (f, 0)),
            pl.BlockSpec((FFB, HIDDEN), lambda f: (f, 0)),
            pl.BlockSpec((HIDDEN, FFB), lambda f: (0, f)),
        ],
        out_specs=pl.BlockSpec((TOK, HIDDEN), lambda f: (0, 0)),
        out_shape=jax.ShapeDtypeStruct((TOK, HIDDEN), jnp.float32),
        scratch_shapes=[pltpu.VMEM((TOK, HIDDEN), jnp.float32)],
        compiler_params=pltpu.CompilerParams(
            vmem_limit_bytes=128 * 1024 * 1024),
    )(hs, shared_gate_w, shared_up_w, shared_down_w)

    pos1f = pos1.reshape(TOK)
    pos2f = pos2.reshape(TOK)
    xg = _make_dispatch()(hs, pos1f, pos2f)

    yrows = pl.pallas_call(
        _gmm_body,
        grid_spec=pltpu.PrefetchScalarGridSpec(
            num_scalar_prefetch=1,
            grid=(NBLK,),
            in_specs=[
                pl.BlockSpec((BM, HIDDEN), lambda b, be: (b, 0)),
                pl.BlockSpec((1, FF, HIDDEN), lambda b, be: (be[b], 0, 0)),
                pl.BlockSpec((1, FF, HIDDEN), lambda b, be: (be[b], 0, 0)),
                pl.BlockSpec((1, HIDDEN, FF), lambda b, be: (be[b], 0, 0)),
            ],
            out_specs=pl.BlockSpec((BM, HIDDEN), lambda b, be: (b, 0)),
        ),
        out_shape=jax.ShapeDtypeStruct((NPAD, HIDDEN), jnp.float32),
        compiler_params=pltpu.CompilerParams(
            vmem_limit_bytes=128 * 1024 * 1024),
    )(be.reshape(NBLKP), xg, gate_w, up_w, down_w)

    out = _make_combine()(base, yrows, pos1f, pos2f, s1b, s2b)

    return out, scores


# speedup vs baseline: 1.3415x; 1.3415x over previous
import jax
import jax.numpy as jnp
from jax import lax
from jax.experimental import pallas as pl
from jax.experimental.pallas import tpu as pltpu
from jax.experimental.pallas import tpu_sc as plsc

HIDDEN = 1024
FF = 2048
E = 8
TOK = 2048
FFB = 512
NFF = FF // FFB

BM = 256
BM_SHIFT = 8
NBLK = (2 * TOK) // BM + E - 1
NBLKP = 32
NPAD = NBLK * BM

NC, NS = 2, 16
NW = NC * NS
TPW = TOK // NW
CH = 16


def _nt(a, b):
    return lax.dot_general(a, b, (((1,), (1,)), ((), ())),
                           preferred_element_type=jnp.float32)


def _top2(logits):
    ii = lax.broadcasted_iota(jnp.int32, logits.shape, 1)
    m1 = jnp.max(logits, axis=1, keepdims=True)
    idx1 = jnp.min(jnp.where(logits == m1, ii, E), axis=1, keepdims=True)
    mask1 = ii == idx1
    rest = jnp.where(mask1, -jnp.inf, logits)
    m2 = jnp.max(rest, axis=1, keepdims=True)
    idx2 = jnp.min(jnp.where(rest == m2, ii, E), axis=1, keepdims=True)
    mask2 = ii == idx2
    return mask1, mask2


def _meta_body(hs_ref, rw_ref, scores_ref, s1b_ref, s2b_ref,
               pos1_ref, pos2_ref, be_ref):
    x = hs_ref[...]
    logits = _nt(x, rw_ref[...])
    mask1, mask2 = _top2(logits)
    topmask = mask1 | mask2
    scores = jnp.where(topmask, jax.nn.sigmoid(logits), 0.0)
    scores_ref[...] = scores

    s1 = jnp.sum(jnp.where(mask1, scores, 0.0), axis=1, keepdims=True)
    s2 = jnp.sum(jnp.where(mask2, scores, 0.0), axis=1, keepdims=True)
    s1b_ref[...] = jnp.broadcast_to(s1, (TOK, 16))
    s2b_ref[...] = jnp.broadcast_to(s2, (TOK, 16))

    tm = topmask.astype(jnp.float32)
    ri = lax.broadcasted_iota(jnp.int32, (TOK, TOK), 0)
    ci = lax.broadcasted_iota(jnp.int32, (TOK, TOK), 1)
    tri = (ci < ri).astype(jnp.float32)
    ranks = lax.dot_general(tri, tm, (((1,), (0,)), ((), ())),
                            preferred_element_type=jnp.float32)

    counts = jnp.sum(tm, axis=0, keepdims=True).astype(jnp.int32)
    nblk_e = (counts + (BM - 1)) >> BM_SHIFT
    ri8 = lax.broadcasted_iota(jnp.int32, (E, E), 0)
    ci8 = lax.broadcasted_iota(jnp.int32, (E, E), 1)
    tri8 = (ri8 < ci8).astype(jnp.float32)
    po_blk = lax.dot_general(nblk_e.astype(jnp.float32), tri8,
                             (((1,), (0,)), ((), ())),
                             preferred_element_type=jnp.float32)
    po_rows = po_blk * float(BM)

    val = po_rows + ranks
    pos1 = jnp.sum(jnp.where(mask1, val, 0.0), axis=1, keepdims=True)
    pos2 = jnp.sum(jnp.where(mask2, val, 0.0), axis=1, keepdims=True)
    pos1_ref[...] = pos1.astype(jnp.int32)
    pos2_ref[...] = pos2.astype(jnp.int32)

    nb = jnp.sum(nblk_e, axis=1, keepdims=True)
    po_i = po_blk.astype(jnp.int32)
    bb = lax.broadcasted_iota(jnp.int32, (NBLKP, E), 0)
    be_raw = jnp.sum((po_i <= bb).astype(jnp.int32), axis=1,
                     keepdims=True) - 1
    be_raw = jnp.maximum(be_raw, 0)
    ie = lax.broadcasted_iota(jnp.int32, (1, E), 1)
    laste = jnp.max(jnp.where(nblk_e > 0, ie, 0), axis=1, keepdims=True)
    rowi = lax.broadcasted_iota(jnp.int32, (NBLKP, 1), 0)
    be = jnp.where(rowi < nb, be_raw, laste)
    be = jnp.where(rowi == NBLKP - 1, nb, be)
    be_ref[...] = be


def _shared_body(hs_ref, sg_ref, su_ref, sd_ref, base_ref, acc_ref):
    f = pl.program_id(0)
    x = hs_ref[...]
    h = jax.nn.silu(_nt(x, sg_ref[...])) * _nt(x, su_ref[...])
    part = _nt(h, sd_ref[...])

    @pl.when(f == 0)
    def _():
        acc_ref[...] = part

    @pl.when(f > 0)
    def _():
        acc_ref[...] += part

    @pl.when(f == NFF - 1)
    def _():
        base_ref[...] = acc_ref[...]


def _gmm_body(be_ref, xg_ref, wg_ref, wu_ref, wd_ref, y_ref):
    b = pl.program_id(0)

    @pl.when(b < be_ref[NBLKP - 1])
    def _():
        x = xg_ref[...]
        h = jax.nn.silu(_nt(x, wg_ref[0])) * _nt(x, wu_ref[0])
        y_ref[...] = _nt(h, wd_ref[0])


def _dispatch_body(hs_hbm, pos1_hbm, pos2_hbm, xg_hbm,
                   p1v, p2v, xrows, sem):
    wid = lax.axis_index("s") * NC + lax.axis_index("c")
    base = wid * TPW
    pltpu.sync_copy(pos1_hbm.at[pl.ds(base, TPW)], p1v)
    pltpu.sync_copy(pos2_hbm.at[pl.ds(base, TPW)], p2v)
    pltpu.sync_copy(hs_hbm.at[pl.ds(base, TPW)], xrows)
    c1 = pltpu.async_copy(xrows, xg_hbm.at[p1v], sem)
    c2 = pltpu.async_copy(xrows, xg_hbm.at[p2v], sem)
    c1.wait()
    c2.wait()


def _combine_body(base_hbm, y_hbm, pos1_hbm, pos2_hbm, s1b_hbm, s2b_hbm,
                  out_hbm, p1v, p2v, y1, y2, bb, s1v, s2v, sem, osem):
    wid = lax.axis_index("s") * NC + lax.axis_index("c")
    nch = TPW // CH

    def fetch(c, slot):
        tok0 = wid * TPW + c * CH
        pltpu.sync_copy(pos1_hbm.at[pl.ds(tok0, CH)], p1v.at[slot])
        pltpu.sync_copy(pos2_hbm.at[pl.ds(tok0, CH)], p2v.at[slot])
        g1 = pltpu.async_copy(y_hbm.at[p1v.at[slot]], y1.at[slot], sem)
        g2 = pltpu.async_copy(y_hbm.at[p2v.at[slot]], y2.at[slot], sem)
        gb = pltpu.async_copy(base_hbm.at[pl.ds(tok0, CH)], bb.at[slot], sem)
        pltpu.sync_copy(s1b_hbm.at[pl.ds(tok0, CH)], s1v.at[slot])
        pltpu.sync_copy(s2b_hbm.at[pl.ds(tok0, CH)], s2v.at[slot])
        return g1, g2, gb

    pend = fetch(0, 0)
    owaits = []
    for c in range(nch):
        slot = c % 2
        for d in pend:
            d.wait()
        if c + 1 < nch:
            nxt = fetch(c + 1, (c + 1) % 2)
        y1s, y2s, bbs = y1.at[slot], y2.at[slot], bb.at[slot]
        s1s, s2s = s1v.at[slot], s2v.at[slot]

        def tbody(t, carry):
            s1 = s1s[t, :]
            s2 = s2s[t, :]
            for j in range(HIDDEN // 16):
                sl = pl.ds(j * 16, 16)
                bbs[t, sl] = bbs[t, sl] + s1 * y1s[t, sl] + s2 * y2s[t, sl]
            return carry

        lax.fori_loop(0, CH, tbody, 0)
        tok0 = wid * TPW + c * CH
        owaits.append(pltpu.async_copy(bb.at[slot], out_hbm.at[pl.ds(tok0, CH)], osem))
        if len(owaits) >= 2:
            owaits.pop(0).wait()
        if c + 1 < nch:
            pend = nxt
    for d in owaits:
        d.wait()


def _sc_mesh():
    return plsc.VectorSubcoreMesh(core_axis_name="c", subcore_axis_name="s",
                                  num_cores=NC, num_subcores=NS)


def _make_dispatch():
    return pl.kernel(
        _dispatch_body,
        out_type=jax.ShapeDtypeStruct((NPAD, HIDDEN), jnp.float32),
        mesh=_sc_mesh(),
        scratch_types=[
            pltpu.VMEM((TPW,), jnp.int32),
            pltpu.VMEM((TPW,), jnp.int32),
            pltpu.VMEM((TPW, HIDDEN), jnp.float32),
            pltpu.SemaphoreType.DMA,
        ],
    )


def _make_combine():
    return pl.kernel(
        _combine_body,
        out_type=jax.ShapeDtypeStruct((TOK, HIDDEN), jnp.float32),
        mesh=_sc_mesh(),
        scratch_types=[
            pltpu.VMEM((2, CH), jnp.int32),
            pltpu.VMEM((2, CH), jnp.int32),
            pltpu.VMEM((2, CH, HIDDEN), jnp.float32),
            pltpu.VMEM((2, CH, HIDDEN), jnp.float32),
            pltpu.VMEM((2, CH, HIDDEN), jnp.float32),
            pltpu.VMEM((2, CH, 16), jnp.float32),
            pltpu.VMEM((2, CH, 16), jnp.float32),
            pltpu.SemaphoreType.DMA,
            pltpu.SemaphoreType.DMA,
        ],
    )


def kernel(hidden_states, router_w, gate_w, up_w, down_w,
           shared_gate_w, shared_up_w, shared_down_w):
    hs = hidden_states.reshape(-1, HIDDEN)

    scores, s1b, s2b, pos1, pos2, be = pl.pallas_call(
        _meta_body,
        out_shape=[
            jax.ShapeDtypeStruct((TOK, E), jnp.float32),
            jax.ShapeDtypeStruct((TOK, 16), jnp.float32),
            jax.ShapeDtypeStruct((TOK, 16), jnp.float32),
            jax.ShapeDtypeStruct((TOK, 1), jnp.int32),
            jax.ShapeDtypeStruct((TOK, 1), jnp.int32),
            jax.ShapeDtypeStruct((NBLKP, 1), jnp.int32),
        ],
        compiler_params=pltpu.CompilerParams(
            vmem_limit_bytes=128 * 1024 * 1024),
    )(hs, router_w)

    base = pl.pallas_call(
        _shared_body,
        grid=(NFF,),
        in_specs=[
            pl.BlockSpec((TOK, HIDDEN), lambda f: (0, 0)),
            pl.BlockSpec((FFB, HIDDEN), lambda f: (f, 0)),
            pl.BlockSpec((FFB, HIDDEN), lambda f: (f, 0)),
            pl.BlockSpec((HIDDEN, FFB), lambda f: (0, f)),
        ],
        out_specs=pl.BlockSpec((TOK, HIDDEN), lambda f: (0, 0)),
        out_shape=jax.ShapeDtypeStruct((TOK, HIDDEN), jnp.float32),
        scratch_shapes=[pltpu.VMEM((TOK, HIDDEN), jnp.float32)],
        compiler_params=pltpu.CompilerParams(
            vmem_limit_bytes=128 * 1024 * 1024),
    )(hs, shared_gate_w, shared_up_w, shared_down_w)

    pos1f = pos1.reshape(TOK)
    pos2f = pos2.reshape(TOK)
    xg = _make_dispatch()(hs, pos1f, pos2f)

    yrows = pl.pallas_call(
        _gmm_body,
        grid_spec=pltpu.PrefetchScalarGridSpec(
            num_scalar_prefetch=1,
            grid=(NBLK,),
            in_specs=[
                pl.BlockSpec((BM, HIDDEN), lambda b, be: (b, 0)),
                pl.BlockSpec((1, FF, HIDDEN), lambda b, be: (be[b], 0, 0)),
                pl.BlockSpec((1, FF, HIDDEN), lambda b, be: (be[b], 0, 0)),
                pl.BlockSpec((1, HIDDEN, FF), lambda b, be: (be[b], 0, 0)),
            ],
            out_specs=pl.BlockSpec((BM, HIDDEN), lambda b, be: (b, 0)),
        ),
        out_shape=jax.ShapeDtypeStruct((NPAD, HIDDEN), jnp.float32),
        compiler_params=pltpu.CompilerParams(
            vmem_limit_bytes=128 * 1024 * 1024),
    )(be.reshape(NBLKP), xg, gate_w, up_w, down_w)

    out = _make_combine()(base, yrows, pos1f, pos2f, s1b, s2b)

    return out, scores
